# Initial kernel scaffold; baseline (speedup 1.0000x reference)
import jax
import jax.numpy as jnp
from jax.experimental import pallas as pl


def kernel(x, edge_index, edge_attr, W1, b1, W2, b2):
    raise NotImplementedError("write your pallas kernel here")


# speedup vs baseline: 12.7161x; 12.7161x over previous
import functools

import jax
import jax.numpy as jnp
from jax import lax
from jax.experimental import pallas as pl
from jax.experimental.pallas import tpu as pltpu
from jax.experimental.pallas import tpu_sc as plsc

N = 10000
D = 128
E = 320000
NC = 2
NS = 16
NW = NC * NS
EPW = E // NW
B = 80
NB = EPW // B

_mesh = plsc.VectorSubcoreMesh(core_axis_name="c", subcore_axis_name="s")


@functools.partial(
    pl.kernel,
    out_type=jax.ShapeDtypeStruct((NC * N,), jnp.float32),
    mesh=_mesh,
    scratch_types=[
        pltpu.VMEM((NB, B), jnp.int32),
        pltpu.VMEM((NB, B), jnp.float32),
        pltpu.VMEM((1000,), jnp.float32),
        pltpu.VMEM_SHARED((N,), jnp.float32),
    ],
    compiler_params=pltpu.CompilerParams(needs_layout_passes=False),
)
def _sc_deg(dst_hbm, w_hbm, zeros_hbm, out_hbm, dst_v, w_v, bounce_v, deg_sh):
    c = lax.axis_index("c")
    s = lax.axis_index("s")
    wid = s * NC + c

    @pl.when(s < 10)
    def _():
        sl = pl.ds(s * 1000, 1000)
        pltpu.sync_copy(zeros_hbm.at[sl], bounce_v)
        pltpu.sync_copy(bounce_v, deg_sh.at[sl])

    pltpu.sync_copy(dst_hbm.at[wid], dst_v)
    pltpu.sync_copy(w_hbm.at[wid], w_v)
    plsc.subcore_barrier()

    def body(i, carry):
        pltpu.sync_copy(w_v.at[i], deg_sh.at[dst_v.at[i]], add=True)
        return carry

    lax.fori_loop(0, NB, body, 0)
    plsc.subcore_barrier()

    @pl.when(s < 10)
    def _():
        pltpu.sync_copy(deg_sh.at[pl.ds(s * 1000, 1000)], bounce_v)
        pltpu.sync_copy(bounce_v, out_hbm.at[pl.ds(c * N + s * 1000, 1000)])


CB = 25
NCH = NB // CB
EPC = CB * B


@functools.partial(
    pl.kernel,
    out_type=jax.ShapeDtypeStruct((NC, N, D), jnp.float32),
    mesh=_mesh,
    scratch_types=[
        pltpu.VMEM((CB, B), jnp.int32),
        pltpu.VMEM((CB, B), jnp.int32),
        pltpu.VMEM((EPC,), jnp.float32),
        pltpu.VMEM((B, D), jnp.float32),
        pltpu.VMEM((40, D), jnp.float32),
        pltpu.VMEM_SHARED((N, D), jnp.float32),
    ],
    compiler_params=pltpu.CompilerParams(needs_layout_passes=False),
)
def _sc_agg(hp_hbm, src_hbm, dst_hbm, wflat_hbm, zeros_hbm, out_hbm,
            src_v, dst_v, w_v, rows_v, bounce_v, acc_sh):
    c = lax.axis_index("c")
    s = lax.axis_index("s")
    wid = s * NC + c
    r0 = s * 1000

    @pl.when(s < 10)
    def _():
        def initk(k, carry):
            sl = pl.ds(r0 + k * 40, 40)

            @pl.when(c == 0)
            def _():
                pltpu.sync_copy(hp_hbm.at[sl], bounce_v)

            @pl.when(c != 0)
            def _():
                pltpu.sync_copy(zeros_hbm.at[sl], bounce_v)

            pltpu.sync_copy(bounce_v, acc_sh.at[sl])
            return carry

        lax.fori_loop(0, 25, initk, 0)

    plsc.subcore_barrier()

    def chunk_body(k, carry):
        row = wid * NCH + k
        pltpu.sync_copy(src_hbm.at[row], src_v)
        pltpu.sync_copy(dst_hbm.at[row], dst_v)
        pltpu.sync_copy(wflat_hbm.at[row], w_v)

        def body(j, carry2):
            pltpu.sync_copy(hp_hbm.at[src_v.at[j]], rows_v)

            def scale(e, c2):
                w16 = plsc.load_gather(
                    w_v, [jnp.full((16,), j * B + e, jnp.int32)]
                )
                for d in range(D // 16):
                    sl = pl.ds(d * 16, 16)
                    rows_v[e, sl] = rows_v[e, sl] * w16
                return c2

            lax.fori_loop(0, B, scale, 0)
            pltpu.sync_copy(rows_v, acc_sh.at[dst_v.at[j]], add=True)
            return carry2

        lax.fori_loop(0, CB, body, 0)
        return carry

    lax.fori_loop(0, NCH, chunk_body, 0)
    plsc.subcore_barrier()

    @pl.when(s < 10)
    def _():
        def outk(k, carry):
            sl = pl.ds(r0 + k * 40, 40)
            pltpu.sync_copy(acc_sh.at[sl], bounce_v)
            pltpu.sync_copy(bounce_v, out_hbm.at[c, sl])
            return carry

        lax.fori_loop(0, 25, outk, 0)


_BN = 1000


def _tc_lin_first(x, W, deg3):
    def body(x_ref, w_ref, deg_ref, hp_ref, dinv_ref):
        d = deg_ref[0] + deg_ref[1] + 1.0
        dinv = jnp.where(d > 0, lax.rsqrt(jnp.maximum(d, 1e-12)), 0.0)
        dinv_ref[...] = dinv
        h = jnp.dot(x_ref[...], w_ref[...], preferred_element_type=jnp.float32)
        hp_ref[...] = h * dinv

    return pl.pallas_call(
        body,
        grid=(N // _BN,),
        in_specs=[
            pl.BlockSpec((_BN, D), lambda i: (i, 0)),
            pl.BlockSpec((D, D), lambda i: (0, 0)),
            pl.BlockSpec((2, _BN, 1), lambda i: (0, i, 0)),
        ],
        out_specs=[
            pl.BlockSpec((_BN, D), lambda i: (i, 0)),
            pl.BlockSpec((_BN, 1), lambda i: (i, 0)),
        ],
        out_shape=[
            jax.ShapeDtypeStruct((N, D), jnp.float32),
            jax.ShapeDtypeStruct((N, 1), jnp.float32),
        ],
    )(x, W, deg3)


def _tc_mid(P, dinv, b, W):
    def body(p_ref, dinv_ref, b_ref, w_ref, out_ref):
        z = jax.nn.relu(dinv_ref[...] * (p_ref[0] + p_ref[1]) + b_ref[...])
        out_ref[...] = (
            jnp.dot(z, w_ref[...], preferred_element_type=jnp.float32)
            * dinv_ref[...]
        )

    return pl.pallas_call(
        body,
        grid=(N // _BN,),
        in_specs=[
            pl.BlockSpec((2, _BN, D), lambda i: (0, i, 0)),
            pl.BlockSpec((_BN, 1), lambda i: (i, 0)),
            pl.BlockSpec((1, D), lambda i: (0, 0)),
            pl.BlockSpec((D, D), lambda i: (0, 0)),
        ],
        out_specs=pl.BlockSpec((_BN, D), lambda i: (i, 0)),
        out_shape=jax.ShapeDtypeStruct((N, D), jnp.float32),
    )(P, dinv, b, W)


def _tc_final(P, dinv, b):
    def body(p_ref, dinv_ref, b_ref, out_ref):
        out_ref[...] = jax.nn.relu(
            dinv_ref[...] * (p_ref[0] + p_ref[1]) + b_ref[...]
        )

    return pl.pallas_call(
        body,
        grid=(N // _BN,),
        in_specs=[
            pl.BlockSpec((2, _BN, D), lambda i: (0, i, 0)),
            pl.BlockSpec((_BN, 1), lambda i: (i, 0)),
            pl.BlockSpec((1, D), lambda i: (0, 0)),
        ],
        out_specs=pl.BlockSpec((_BN, D), lambda i: (i, 0)),
        out_shape=jax.ShapeDtypeStruct((N, D), jnp.float32),
    )(P, dinv, b)


@jax.jit
def kernel(x, edge_index, edge_attr, W1, b1, W2, b2):
    dst2 = edge_index[1].reshape(NW, NB, B)
    w2 = edge_attr.reshape(NW, NB, B)
    src3 = edge_index[0].reshape(NW * NCH, CB, B)
    dst3 = edge_index[1].reshape(NW * NCH, CB, B)
    wf = edge_attr.reshape(NW * NCH, EPC)
    zeros1 = jnp.zeros((N,), jnp.float32)
    zeros2 = jnp.zeros((N, D), jnp.float32)

    deg2 = _sc_deg(dst2, w2, zeros1)
    hp1, dinv = _tc_lin_first(x, W1, deg2.reshape(NC, N, 1))
    P1 = _sc_agg(hp1, src3, dst3, wf, zeros2)
    hp2 = _tc_mid(P1, dinv, b1.reshape(1, D), W2)
    P2 = _sc_agg(hp2, src3, dst3, wf, zeros2)
    return _tc_final(P2, dinv, b2.reshape(1, D))
